# baseline (speedup 1.0000x reference)
import functools

import jax
import jax.numpy as jnp
from jax import lax
from jax.experimental import pallas as pl
from jax.experimental.pallas import tpu as pltpu
from jax.experimental.pallas import tpu_sc as plsc

_NC = 2
_NS = 16
_NW = _NC * _NS


def _argmax_body(s_ref, a_ref, o_ref, *, n_actions):
    vx = s_ref[:, 0, :]
    vy = s_ref[:, 1, :]
    blk, e = vx.shape
    iota = lax.broadcasted_iota(jnp.int32, (blk, e), 1)
    mx = jnp.max(vx, axis=-1, keepdims=True)
    amx = jnp.min(jnp.where(vx == mx, iota, e), axis=-1)
    my = jnp.max(vy, axis=-1, keepdims=True)
    amy = jnp.min(jnp.where(vy == my, iota, e), axis=-1)
    o_ref[0, 0, :] = (
        amx * (e * n_actions)
        + (amy >> 7) * (n_actions * 128)
        + a_ref[0, 0, :] * 128
        + (amy & 127)
    )


def _make_sc_gather(n_elems, b_per_w, n_idx_chunks, idx_chunk):
    mesh = plsc.VectorSubcoreMesh(core_axis_name="c", subcore_axis_name="s")

    @functools.partial(
        pl.kernel,
        mesh=mesh,
        out_type=jax.ShapeDtypeStruct((_NW * b_per_w,), jnp.float32),
        scratch_types=[
            pltpu.VMEM((b_per_w,), jnp.int32),
            pltpu.VMEM((b_per_w,), jnp.float32),
            pltpu.SemaphoreType.DMA,
        ],
        compiler_params=pltpu.CompilerParams(use_tc_tiling_on_sc=True),
    )
    def sc_gather(table_hbm, idx_hbm, out_hbm, idx_v, vals_v, sem):
        wid = lax.axis_index("s") * _NC + lax.axis_index("c")
        base = wid * b_per_w
        pltpu.sync_copy(idx_hbm.at[pl.ds(base, b_per_w)], idx_v)
        handles = [
            pltpu.async_copy(
                table_hbm.at[idx_v.at[pl.ds(g * idx_chunk, idx_chunk)]],
                vals_v.at[pl.ds(g * idx_chunk, idx_chunk)],
                sem,
            )
            for g in range(n_idx_chunks)
        ]
        for h in handles:
            h.wait()
        pltpu.sync_copy(vals_v, out_hbm.at[pl.ds(base, b_per_w)])

    return sc_gather


def kernel(s, a, env_size, table):
    B, _, E = s.shape
    A = table.shape[-1]
    BLK = 512
    n_blocks = B // BLK

    idx3 = pl.pallas_call(
        functools.partial(_argmax_body, n_actions=A),
        grid=(n_blocks,),
        in_specs=[
            pl.BlockSpec((BLK, 2, E), lambda i: (i, 0, 0)),
            pl.BlockSpec((1, 1, BLK), lambda i: (i, 0, 0)),
        ],
        out_specs=pl.BlockSpec((1, 1, BLK), lambda i: (i, 0, 0)),
        out_shape=jax.ShapeDtypeStruct((n_blocks, 1, BLK), jnp.int32),
    )(s, a.reshape(n_blocks, 1, BLK))

    b_per_w = B // _NW
    idx_chunk = 128
    n_idx_chunks = b_per_w // idx_chunk

    table_flat = (
        table.reshape(E, E // 128, 128, A).transpose(0, 1, 3, 2).reshape(E * E * A)
    )
    idx_flat = idx3.reshape(B)

    sc_gather = _make_sc_gather(E * E * A, b_per_w, n_idx_chunks, idx_chunk)
    return sc_gather(table_flat, idx_flat)


# speedup vs baseline: 1.0118x; 1.0118x over previous
import functools

import jax
import jax.numpy as jnp
from jax import lax
from jax.experimental import pallas as pl
from jax.experimental.pallas import tpu as pltpu
from jax.experimental.pallas import tpu_sc as plsc

_NC = 2
_NS = 16
_NW = _NC * _NS


def _argmax_body(s_ref, a_ref, o_ref, *, n_actions):
    vx = s_ref[:, 0, :]
    vy = s_ref[:, 1, :]
    blk, e = vx.shape
    iota = lax.broadcasted_iota(jnp.int32, (blk, e), 1)
    mx = jnp.max(vx, axis=-1, keepdims=True)
    amx = jnp.min(jnp.where(vx == mx, iota, e), axis=-1)
    my = jnp.max(vy, axis=-1, keepdims=True)
    amy = jnp.min(jnp.where(vy == my, iota, e), axis=-1)
    o_ref[0, 0, :] = (
        amx * (e * n_actions)
        + (amy >> 7) * (n_actions * 128)
        + a_ref[0, 0, :] * 128
        + (amy & 127)
    )


def _make_sc_gather(n_elems, b_per_w, n_idx_chunks, idx_chunk):
    mesh = plsc.VectorSubcoreMesh(core_axis_name="c", subcore_axis_name="s")

    @functools.partial(
        pl.kernel,
        mesh=mesh,
        out_type=jax.ShapeDtypeStruct((_NW * b_per_w,), jnp.float32),
        scratch_types=[
            pltpu.VMEM((b_per_w,), jnp.int32),
            pltpu.VMEM((b_per_w,), jnp.float32),
            pltpu.SemaphoreType.DMA,
        ],
        compiler_params=pltpu.CompilerParams(use_tc_tiling_on_sc=True),
    )
    def sc_gather(table_hbm, idx_hbm, out_hbm, idx_v, vals_v, sem):
        wid = lax.axis_index("s") * _NC + lax.axis_index("c")
        base = wid * b_per_w
        pltpu.sync_copy(idx_hbm.at[pl.ds(base, b_per_w)], idx_v)
        handles = [
            pltpu.async_copy(
                table_hbm.at[idx_v.at[pl.ds(g * idx_chunk, idx_chunk)]],
                vals_v.at[pl.ds(g * idx_chunk, idx_chunk)],
                sem,
            )
            for g in range(n_idx_chunks)
        ]
        for h in handles:
            h.wait()
        pltpu.sync_copy(vals_v, out_hbm.at[pl.ds(base, b_per_w)])

    return sc_gather


def kernel(s, a, env_size, table):
    B, _, E = s.shape
    A = table.shape[-1]
    BLK = 1024
    n_blocks = B // BLK

    idx3 = pl.pallas_call(
        functools.partial(_argmax_body, n_actions=A),
        grid=(n_blocks,),
        in_specs=[
            pl.BlockSpec((BLK, 2, E), lambda i: (i, 0, 0)),
            pl.BlockSpec((1, 1, BLK), lambda i: (i, 0, 0)),
        ],
        out_specs=pl.BlockSpec((1, 1, BLK), lambda i: (i, 0, 0)),
        out_shape=jax.ShapeDtypeStruct((n_blocks, 1, BLK), jnp.int32),
    )(s, a.reshape(n_blocks, 1, BLK))

    b_per_w = B // _NW
    idx_chunk = 128
    n_idx_chunks = b_per_w // idx_chunk

    table_flat = (
        table.reshape(E, E // 128, 128, A).transpose(0, 1, 3, 2).reshape(E * E * A)
    )
    idx_flat = idx3.reshape(B)

    sc_gather = _make_sc_gather(E * E * A, b_per_w, n_idx_chunks, idx_chunk)
    return sc_gather(table_flat, idx_flat)
